# baseline (speedup 1.0000x reference)
import functools

import jax
import jax.numpy as jnp
import numpy as np
from jax import lax
from jax.experimental import pallas as pl
from jax.experimental.pallas import tpu as pltpu
from jax.experimental.pallas import tpu_sc as plsc

N = 10000
E = 160000
DIN = 42
DP = 48
DE = 16
HID = 16
WNUM = 468

NC, NS = 2, 16
NW = NC * NS
CH = 128
NCHT = E // CH
TPW = 40
EPAD = TPW * NW * CH
NPAD = 10240
RPS = NPAD // NS

BE = 6400
BS = BE // 128
GRID = E // BE

_INV3 = float(1.0 / np.sqrt(3.0))
_INV2 = float(1.0 / np.sqrt(2.0))


def _gather_body(table_hbm, idx_hbm, out_hbm, slab_v, rows_a, rows_b,
                 sem_a, sem_b):
  c = lax.axis_index("c")
  s = lax.axis_index("s")
  wid = s * NC + c
  pltpu.sync_copy(idx_hbm.at[:, wid], slab_v)

  def issue(t, buf, sem):
    @pl.when(t * NW + wid < NCHT)
    def _():
      pltpu.async_copy(table_hbm.at[slab_v.at[t]], buf, sem)

  def drain(t, buf, sem):
    g = t * NW + wid

    @pl.when(g < NCHT)
    def _():
      pltpu.make_async_copy(table_hbm.at[slab_v.at[t]], buf, sem).wait()
      pltpu.sync_copy(buf, out_hbm.at[pl.ds(g * CH, CH)])

  def step(j, carry):
    issue(2 * j, rows_a, sem_a)
    issue(2 * j + 1, rows_b, sem_b)
    drain(2 * j, rows_a, sem_a)
    drain(2 * j + 1, rows_b, sem_b)
    return carry

  lax.fori_loop(0, TPW // 2, step, 0)


def _sc_gather(table, dst3):
  mesh = plsc.VectorSubcoreMesh(core_axis_name="c", subcore_axis_name="s")
  return pl.kernel(
      _gather_body,
      out_type=jax.ShapeDtypeStruct((E, DP), jnp.float32),
      mesh=mesh,
      compiler_params=pltpu.CompilerParams(use_tc_tiling_on_sc=False),
      scratch_types=[
          pltpu.VMEM((TPW, CH), jnp.int32),
          pltpu.VMEM((CH, DP), jnp.float32),
          pltpu.VMEM((CH, DP), jnp.float32),
          pltpu.SemaphoreType.DMA,
          pltpu.SemaphoreType.DMA,
      ],
  )(table, dst3)


def _scatter_body(tp_hbm, src_hbm, zsum_hbm, psum_hbm,
                  slab_v, rows_a, rows_b, zb_v, ssum, sem_a, sem_b):
  c = lax.axis_index("c")
  s = lax.axis_index("s")
  wid = s * NC + c

  pltpu.sync_copy(zsum_hbm, zb_v)
  pltpu.sync_copy(zb_v, ssum.at[pl.ds(s * RPS, RPS)])
  pltpu.sync_copy(src_hbm.at[:, wid], slab_v)
  plsc.subcore_barrier()

  def issue(t, buf, sem):
    g = t * NW + wid

    @pl.when(g < NCHT)
    def _():
      pltpu.async_copy(tp_hbm.at[pl.ds(g * CH, CH)], buf, sem)

  def drain(t, buf, sem):
    g = t * NW + wid

    @pl.when(g < NCHT)
    def _():
      pltpu.make_async_copy(tp_hbm.at[pl.ds(g * CH, CH)], buf, sem).wait()
      pltpu.sync_copy(buf, ssum.at[slab_v.at[t]], add=True)

  def step(j, carry):
    issue(2 * j, rows_a, sem_a)
    issue(2 * j + 1, rows_b, sem_b)
    drain(2 * j, rows_a, sem_a)
    drain(2 * j + 1, rows_b, sem_b)
    return carry

  lax.fori_loop(0, TPW // 2, step, 0)
  plsc.subcore_barrier()

  pltpu.sync_copy(ssum.at[pl.ds(s * RPS, RPS)], zb_v)
  pltpu.sync_copy(zb_v, psum_hbm.at[c, pl.ds(s * RPS, RPS)])


def _sc_scatter(tp, src3, zsum):
  mesh = plsc.VectorSubcoreMesh(core_axis_name="c", subcore_axis_name="s")
  return pl.kernel(
      _scatter_body,
      out_type=jax.ShapeDtypeStruct((NC, NPAD, DP), jnp.float32),
      mesh=mesh,
      compiler_params=pltpu.CompilerParams(use_tc_tiling_on_sc=False),
      scratch_types=[
          pltpu.VMEM((TPW, CH), jnp.int32),
          pltpu.VMEM((CH, DP), jnp.float32),
          pltpu.VMEM((CH, DP), jnp.float32),
          pltpu.VMEM((RPS, DP), jnp.float32),
          pltpu.VMEM_SHARED((NPAD, DP), jnp.float32),
          pltpu.SemaphoreType.DMA,
          pltpu.SemaphoreType.DMA,
      ],
  )(tp, src3, zsum)


def _dense_body(ea_ref, sh_ref, x_ref, w1t_ref, b1_ref, w2t_ref, b2_ref,
                eye_ref, tp_ref):
  f32 = jnp.float32
  eye = eye_ref[...]
  eaT = lax.dot_general(eye[:DE, :DE], ea_ref[...], (((1,), (1,)), ((), ())),
                        preferred_element_type=f32)
  h = jnp.maximum(
      lax.dot_general(w1t_ref[...], eaT, (((1,), (0,)), ((), ())),
                      preferred_element_type=f32) + b1_ref[...], 0.0)
  w2d = lax.dot_general(w2t_ref[...], h, (((1,), (0,)), ((), ())),
                        preferred_element_type=f32) + b2_ref[...]
  shT = lax.dot_general(eye[:4, :4], sh_ref[...], (((1,), (1,)), ((), ())),
                        preferred_element_type=f32)
  x = x_ref[...]
  xt = lax.dot_general(eye, x, (((1,), (1,)), ((), ())),
                       preferred_element_type=f32)

  X = xt.reshape(DP, BS, 128)
  S = shT.reshape(4, BS, 128)
  W = w2d.reshape(WNUM, BS, 128)
  SH0 = S[0]
  SH1 = [S[1], S[2], S[3]]

  def XP(j):
    return X[j]

  def WP(r):
    return W[r]

  f0e = [XP(i) * SH0 for i in range(16)]
  f0e += [(XP(16 + 3 * i) * SH1[0] + XP(17 + 3 * i) * SH1[1]
           + XP(18 + 3 * i) * SH1[2]) * _INV3 for i in range(4)]

  o1o = [[XP(i) * SH1[cc] for cc in range(3)] for i in range(16)]
  o1o += [[XP(16 + 3 * i + cc) * SH0 for cc in range(3)] for i in range(4)]
  for i in range(4):
    a = [XP(28 + 3 * i + cc) for cc in range(3)]
    o1o.append([(a[(cc + 1) % 3] * SH1[(cc + 2) % 3]
                 - a[(cc + 2) % 3] * SH1[(cc + 1) % 3]) * _INV2
                for cc in range(3)])

  o1e = []
  for i in range(4):
    a = [XP(16 + 3 * i + cc) for cc in range(3)]
    o1e.append([(a[(cc + 1) % 3] * SH1[(cc + 2) % 3]
                 - a[(cc + 2) % 3] * SH1[(cc + 1) % 3]) * _INV2
                for cc in range(3)])
  o1e += [[XP(28 + 3 * i + cc) * SH0 for cc in range(3)] for i in range(4)]
  o1e += [[XP(40 + i) * SH1[cc] for cc in range(3)] for i in range(2)]

  f0o = [(XP(28 + 3 * i) * SH1[0] + XP(29 + 3 * i) * SH1[1]
          + XP(30 + 3 * i) * SH1[2]) * _INV3 for i in range(4)]
  f0o += [XP(40 + i) * SH0 for i in range(2)]

  planes = []
  for o in range(16):
    acc = f0e[0] * WP(o)
    for i in range(1, 20):
      acc += f0e[i] * WP(i * 16 + o)
    planes.append(acc)
  for o in range(4):
    for cc in range(3):
      acc = o1o[0][cc] * WP(320 + o)
      for i in range(1, 24):
        acc += o1o[i][cc] * WP(320 + i * 4 + o)
      planes.append(acc)
  for o in range(4):
    for cc in range(3):
      acc = o1e[0][cc] * WP(416 + o)
      for i in range(1, 10):
        acc += o1e[i][cc] * WP(416 + i * 4 + o)
      planes.append(acc)
  for o in range(2):
    acc = f0o[0] * WP(456 + o)
    for i in range(1, 6):
      acc += f0o[i] * WP(456 + i * 2 + o)
    planes.append(acc)

  zero = jnp.zeros_like(planes[0])
  planes += [jnp.ones_like(zero)] + [zero] * (DP - DIN - 1)
  tpt = jnp.stack(planes, axis=0).reshape(DP, BE)
  tp_ref[...] = lax.dot_general(tpt, eye, (((0,), (0,)), ((), ())),
                                preferred_element_type=f32)


def _tc_dense(ea, sh, x, w1t, b1c, w2t, b2c, eye48):
  return pl.pallas_call(
      _dense_body,
      grid=(GRID,),
      in_specs=[
          pl.BlockSpec((BE, DE), lambda i: (i, 0)),
          pl.BlockSpec((BE, 4), lambda i: (i, 0)),
          pl.BlockSpec((BE, DP), lambda i: (i, 0)),
          pl.BlockSpec((DE, DE), lambda i: (0, 0)),
          pl.BlockSpec((DE, 1), lambda i: (0, 0)),
          pl.BlockSpec((WNUM, DE), lambda i: (0, 0)),
          pl.BlockSpec((WNUM, 1), lambda i: (0, 0)),
          pl.BlockSpec((DP, DP), lambda i: (0, 0)),
      ],
      out_specs=pl.BlockSpec((BE, DP), lambda i: (i, 0)),
      out_shape=jax.ShapeDtypeStruct((E, DP), jnp.float32),
  )(ea, sh, x, w1t, b1c, w2t, b2c, eye48)


def _combine_body(ps_ref, na_ref, out_ref):
  sums = ps_ref[0] + ps_ref[1]
  cnt = sums[:, DIN:DIN + 1]
  out_ref[...] = sums[:, :DIN] / jnp.maximum(cnt, 1.0) + na_ref[...]


def _tc_combine(psum, node_attr):
  bn = 1000
  return pl.pallas_call(
      _combine_body,
      grid=(N // bn,),
      in_specs=[
          pl.BlockSpec((NC, bn, DP), lambda i: (0, i, 0)),
          pl.BlockSpec((bn, DIN), lambda i: (i, 0)),
      ],
      out_specs=pl.BlockSpec((bn, DIN), lambda i: (i, 0)),
      out_shape=jax.ShapeDtypeStruct((N, DIN), jnp.float32),
  )(psum, node_attr)


@jax.jit
def kernel(node_attr, edge_index, edge_attr, edge_sh,
           fc_w1, fc_b1, fc_w2, fc_b2):
  f32 = jnp.float32
  node_attr = node_attr.astype(f32)
  edge_src = edge_index[0].astype(jnp.int32)
  edge_dst = edge_index[1].astype(jnp.int32)

  table = jnp.pad(node_attr, ((0, 0), (0, DP - DIN)))
  dst3 = jnp.pad(edge_dst, (0, EPAD - E)).reshape(TPW, NW, CH)
  src3 = jnp.pad(edge_src, (0, EPAD - E)).reshape(TPW, NW, CH)

  scale = np.concatenate([
      np.full(320, 1.0 / np.sqrt(20.0)),
      np.full(96, 1.0 / np.sqrt(24.0)),
      np.full(40, 1.0 / np.sqrt(10.0)),
      np.full(12, 1.0 / np.sqrt(6.0)),
  ]).astype(np.float32)
  w1t = fc_w1.astype(f32).T
  b1c = fc_b1.astype(f32)[:, None]
  w2t = (fc_w2.astype(f32) * scale[None, :]).T
  b2c = (fc_b2.astype(f32) * scale)[:, None]
  eye48 = jnp.eye(DP, dtype=f32)

  x = _sc_gather(table, dst3)
  tp = _tc_dense(edge_attr.astype(f32), edge_sh.astype(f32), x,
                 w1t, b1c, w2t, b2c, eye48)

  zsum = jnp.zeros((RPS, DP), f32)
  psum = _sc_scatter(tp, src3, zsum)

  return _tc_combine(psum, node_attr)


# speedup vs baseline: 1.3863x; 1.3863x over previous
import functools

import jax
import jax.numpy as jnp
import numpy as np
from jax import lax
from jax.experimental import pallas as pl
from jax.experimental.pallas import tpu as pltpu
from jax.experimental.pallas import tpu_sc as plsc

N = 10000
E = 160000
DIN = 42
DP = 48
LW = 128
DE = 16
HID = 16
WNUM = 468

NC, NS = 2, 16
NW = NC * NS
CH = 128
NCHT = E // CH
TPW = 40
EPAD = TPW * NW * CH
NPAD = 10240
RPS = NPAD // NS

BE = 6400
BS = BE // 128
GRID = E // BE

_INV3 = float(1.0 / np.sqrt(3.0))
_INV2 = float(1.0 / np.sqrt(2.0))


def _gather_body(table_hbm, idx_hbm, out_hbm, slab_v, rows_a, rows_b,
                 sem_a, sem_b):
  c = lax.axis_index("c")
  s = lax.axis_index("s")
  wid = s * NC + c
  pltpu.sync_copy(idx_hbm.at[:, wid], slab_v)

  def issue(t, buf, sem):
    @pl.when(t * NW + wid < NCHT)
    def _():
      pltpu.async_copy(table_hbm.at[slab_v.at[t]], buf, sem)

  def drain(t, buf, sem):
    g = t * NW + wid

    @pl.when(g < NCHT)
    def _():
      pltpu.make_async_copy(table_hbm.at[slab_v.at[t]], buf, sem).wait()
      pltpu.sync_copy(buf, out_hbm.at[pl.ds(g * CH, CH), pl.ds(0, DP)])

  def step(j, carry):
    issue(2 * j, rows_a, sem_a)
    issue(2 * j + 1, rows_b, sem_b)
    drain(2 * j, rows_a, sem_a)
    drain(2 * j + 1, rows_b, sem_b)
    return carry

  lax.fori_loop(0, TPW // 2, step, 0)


def _sc_gather(table, dst3):
  mesh = plsc.VectorSubcoreMesh(core_axis_name="c", subcore_axis_name="s")
  return pl.kernel(
      _gather_body,
      out_type=jax.ShapeDtypeStruct((E, LW), jnp.float32),
      mesh=mesh,
      compiler_params=pltpu.CompilerParams(use_tc_tiling_on_sc=False),
      scratch_types=[
          pltpu.VMEM((TPW, CH), jnp.int32),
          pltpu.VMEM((CH, DP), jnp.float32),
          pltpu.VMEM((CH, DP), jnp.float32),
          pltpu.SemaphoreType.DMA,
          pltpu.SemaphoreType.DMA,
      ],
  )(table, dst3)


def _scatter_body(tp_hbm, src_hbm, zsum_hbm, psum_hbm,
                  slab_v, rows_a, rows_b, zb_v, ssum, sem_a, sem_b):
  c = lax.axis_index("c")
  s = lax.axis_index("s")
  wid = s * NC + c

  pltpu.sync_copy(zsum_hbm, zb_v)
  pltpu.sync_copy(zb_v, ssum.at[pl.ds(s * RPS, RPS)])
  pltpu.sync_copy(src_hbm.at[:, wid], slab_v)
  plsc.subcore_barrier()

  def issue(t, buf, sem):
    g = t * NW + wid

    @pl.when(g < NCHT)
    def _():
      pltpu.async_copy(tp_hbm.at[pl.ds(g * CH, CH), pl.ds(0, DP)], buf, sem)

  def drain(t, buf, sem):
    g = t * NW + wid

    @pl.when(g < NCHT)
    def _():
      pltpu.make_async_copy(
          tp_hbm.at[pl.ds(g * CH, CH), pl.ds(0, DP)], buf, sem).wait()
      pltpu.sync_copy(buf, ssum.at[slab_v.at[t]], add=True)

  def step(j, carry):
    issue(2 * j, rows_a, sem_a)
    issue(2 * j + 1, rows_b, sem_b)
    drain(2 * j, rows_a, sem_a)
    drain(2 * j + 1, rows_b, sem_b)
    return carry

  lax.fori_loop(0, TPW // 2, step, 0)
  plsc.subcore_barrier()

  pltpu.sync_copy(ssum.at[pl.ds(s * RPS, RPS)], zb_v)
  pltpu.sync_copy(zb_v, psum_hbm.at[c, pl.ds(s * RPS, RPS), pl.ds(0, DP)])


def _sc_scatter(tp, src3, zsum):
  mesh = plsc.VectorSubcoreMesh(core_axis_name="c", subcore_axis_name="s")
  return pl.kernel(
      _scatter_body,
      out_type=jax.ShapeDtypeStruct((NC, NPAD, LW), jnp.float32),
      mesh=mesh,
      compiler_params=pltpu.CompilerParams(use_tc_tiling_on_sc=False),
      scratch_types=[
          pltpu.VMEM((TPW, CH), jnp.int32),
          pltpu.VMEM((CH, DP), jnp.float32),
          pltpu.VMEM((CH, DP), jnp.float32),
          pltpu.VMEM((RPS, DP), jnp.float32),
          pltpu.VMEM_SHARED((NPAD, DP), jnp.float32),
          pltpu.SemaphoreType.DMA,
          pltpu.SemaphoreType.DMA,
      ],
  )(tp, src3, zsum)


def _dense_body(ea_ref, sh_ref, x_ref, w1t_ref, b1_ref, w2t_ref, b2_ref,
                eyep_ref, tp_ref):
  f32 = jnp.float32
  eyep = eyep_ref[...]
  eaT = lax.dot_general(eyep[:DE, :DE], ea_ref[...], (((1,), (1,)), ((), ())),
                        preferred_element_type=f32)
  h = jnp.maximum(
      lax.dot_general(w1t_ref[...], eaT, (((1,), (0,)), ((), ())),
                      preferred_element_type=f32) + b1_ref[...], 0.0)
  w2d = lax.dot_general(w2t_ref[...], h, (((1,), (0,)), ((), ())),
                        preferred_element_type=f32) + b2_ref[...]
  shT = lax.dot_general(eyep[:4, :4], sh_ref[...], (((1,), (1,)), ((), ())),
                        preferred_element_type=f32)
  lane = lax.broadcasted_iota(jnp.int32, (1, LW), 1)
  xm = jnp.where(lane < DP, x_ref[...], 0.0)
  xt = lax.dot_general(eyep, xm, (((1,), (1,)), ((), ())),
                       preferred_element_type=f32)

  X = xt.reshape(DP, BS, 128)
  S = shT.reshape(4, BS, 128)
  W = w2d.reshape(WNUM, BS, 128)
  SH0 = S[0]
  SH1 = [S[1], S[2], S[3]]

  def XP(j):
    return X[j]

  def WP(r):
    return W[r]

  f0e = [XP(i) * SH0 for i in range(16)]
  f0e += [(XP(16 + 3 * i) * SH1[0] + XP(17 + 3 * i) * SH1[1]
           + XP(18 + 3 * i) * SH1[2]) * _INV3 for i in range(4)]

  o1o = [[XP(i) * SH1[cc] for cc in range(3)] for i in range(16)]
  o1o += [[XP(16 + 3 * i + cc) * SH0 for cc in range(3)] for i in range(4)]
  for i in range(4):
    a = [XP(28 + 3 * i + cc) for cc in range(3)]
    o1o.append([(a[(cc + 1) % 3] * SH1[(cc + 2) % 3]
                 - a[(cc + 2) % 3] * SH1[(cc + 1) % 3]) * _INV2
                for cc in range(3)])

  o1e = []
  for i in range(4):
    a = [XP(16 + 3 * i + cc) for cc in range(3)]
    o1e.append([(a[(cc + 1) % 3] * SH1[(cc + 2) % 3]
                 - a[(cc + 2) % 3] * SH1[(cc + 1) % 3]) * _INV2
                for cc in range(3)])
  o1e += [[XP(28 + 3 * i + cc) * SH0 for cc in range(3)] for i in range(4)]
  o1e += [[XP(40 + i) * SH1[cc] for cc in range(3)] for i in range(2)]

  f0o = [(XP(28 + 3 * i) * SH1[0] + XP(29 + 3 * i) * SH1[1]
          + XP(30 + 3 * i) * SH1[2]) * _INV3 for i in range(4)]
  f0o += [XP(40 + i) * SH0 for i in range(2)]

  planes = []
  for o in range(16):
    acc = f0e[0] * WP(o)
    for i in range(1, 20):
      acc += f0e[i] * WP(i * 16 + o)
    planes.append(acc)
  for o in range(4):
    for cc in range(3):
      acc = o1o[0][cc] * WP(320 + o)
      for i in range(1, 24):
        acc += o1o[i][cc] * WP(320 + i * 4 + o)
      planes.append(acc)
  for o in range(4):
    for cc in range(3):
      acc = o1e[0][cc] * WP(416 + o)
      for i in range(1, 10):
        acc += o1e[i][cc] * WP(416 + i * 4 + o)
      planes.append(acc)
  for o in range(2):
    acc = f0o[0] * WP(456 + o)
    for i in range(1, 6):
      acc += f0o[i] * WP(456 + i * 2 + o)
    planes.append(acc)

  zero = jnp.zeros_like(planes[0])
  planes += [jnp.ones_like(zero)] + [zero] * (DP - DIN - 1)
  tpt = jnp.stack(planes, axis=0).reshape(DP, BE)
  tp_ref[...] = lax.dot_general(tpt, eyep, (((0,), (0,)), ((), ())),
                                preferred_element_type=f32)


def _tc_dense(ea, sh, x, w1t, b1c, w2t, b2c, eyep):
  return pl.pallas_call(
      _dense_body,
      grid=(GRID,),
      in_specs=[
          pl.BlockSpec((BE, DE), lambda i: (i, 0)),
          pl.BlockSpec((BE, 4), lambda i: (i, 0)),
          pl.BlockSpec((BE, LW), lambda i: (i, 0)),
          pl.BlockSpec((DE, DE), lambda i: (0, 0)),
          pl.BlockSpec((DE, 1), lambda i: (0, 0)),
          pl.BlockSpec((WNUM, DE), lambda i: (0, 0)),
          pl.BlockSpec((WNUM, 1), lambda i: (0, 0)),
          pl.BlockSpec((DP, LW), lambda i: (0, 0)),
      ],
      out_specs=pl.BlockSpec((BE, LW), lambda i: (i, 0)),
      out_shape=jax.ShapeDtypeStruct((E, LW), jnp.float32),
  )(ea, sh, x, w1t, b1c, w2t, b2c, eyep)


def _combine_body(ps_ref, na_ref, out_ref):
  sums = ps_ref[0] + ps_ref[1]
  cnt = sums[:N, DIN:DIN + 1]
  out_ref[...] = sums[:N, :DIN] / jnp.maximum(cnt, 1.0) + na_ref[...]


def _tc_combine(psum, node_attr):
  return pl.pallas_call(
      _combine_body,
      out_shape=jax.ShapeDtypeStruct((N, DIN), jnp.float32),
  )(psum, node_attr)


@jax.jit
def kernel(node_attr, edge_index, edge_attr, edge_sh,
           fc_w1, fc_b1, fc_w2, fc_b2):
  f32 = jnp.float32
  node_attr = node_attr.astype(f32)
  edge_src = edge_index[0].astype(jnp.int32)
  edge_dst = edge_index[1].astype(jnp.int32)

  table = jnp.pad(node_attr, ((0, 0), (0, DP - DIN)))
  dst3 = jnp.pad(edge_dst, (0, EPAD - E)).reshape(TPW, NW, CH)
  src3 = jnp.pad(edge_src, (0, EPAD - E)).reshape(TPW, NW, CH)

  scale = np.concatenate([
      np.full(320, 1.0 / np.sqrt(20.0)),
      np.full(96, 1.0 / np.sqrt(24.0)),
      np.full(40, 1.0 / np.sqrt(10.0)),
      np.full(12, 1.0 / np.sqrt(6.0)),
  ]).astype(np.float32)
  w1t = fc_w1.astype(f32).T
  b1c = fc_b1.astype(f32)[:, None]
  w2t = (fc_w2.astype(f32) * scale[None, :]).T
  b2c = (fc_b2.astype(f32) * scale)[:, None]
  eyep = jnp.eye(DP, LW, dtype=f32)

  x = _sc_gather(table, dst3)
  tp = _tc_dense(edge_attr.astype(f32), edge_sh.astype(f32), x,
                 w1t, b1c, w2t, b2c, eyep)

  zsum = jnp.zeros((RPS, DP), f32)
  psum = _sc_scatter(tp, src3, zsum)

  return _tc_combine(psum, node_attr)


# speedup vs baseline: 1.8032x; 1.3007x over previous
import functools

import jax
import jax.numpy as jnp
import numpy as np
from jax import lax
from jax.experimental import pallas as pl
from jax.experimental.pallas import tpu as pltpu
from jax.experimental.pallas import tpu_sc as plsc

N = 10000
E = 160000
DIN = 42
DP = 48
LW = 128
DE = 16
HID = 16
WNUM = 468

NC, NS = 2, 16
NW = NC * NS
CH = 128
NCHT = E // CH
TPW = 40
EPAD = TPW * NW * CH
NPAD = 10240
RPS = NPAD // NS

BE = 6400
BS = BE // 128
GRID = E // BE

_INV3 = float(1.0 / np.sqrt(3.0))
_INV2 = float(1.0 / np.sqrt(2.0))


def _gather_body(table_hbm, idx_hbm, out_hbm, slab_v, rows_a, rows_b,
                 sem_a, sem_b):
  c = lax.axis_index("c")
  s = lax.axis_index("s")
  wid = s * NC + c
  pltpu.sync_copy(idx_hbm.at[:, wid], slab_v)

  def issue(t, buf, sem):
    @pl.when(t * NW + wid < NCHT)
    def _():
      pltpu.async_copy(table_hbm.at[slab_v.at[t]], buf, sem)

  def drain(t, buf, sem):
    g = t * NW + wid

    @pl.when(g < NCHT)
    def _():
      pltpu.make_async_copy(table_hbm.at[slab_v.at[t]], buf, sem).wait()
      pltpu.sync_copy(buf, out_hbm.at[pl.ds(g * CH, CH), pl.ds(0, DP)])

  def step(j, carry):
    issue(2 * j, rows_a, sem_a)
    issue(2 * j + 1, rows_b, sem_b)
    drain(2 * j, rows_a, sem_a)
    drain(2 * j + 1, rows_b, sem_b)
    return carry

  lax.fori_loop(0, TPW // 2, step, 0)


def _sc_gather(table, dst3):
  mesh = plsc.VectorSubcoreMesh(core_axis_name="c", subcore_axis_name="s")
  return pl.kernel(
      _gather_body,
      out_type=jax.ShapeDtypeStruct((E, LW), jnp.float32),
      mesh=mesh,
      compiler_params=pltpu.CompilerParams(use_tc_tiling_on_sc=False),
      scratch_types=[
          pltpu.VMEM((TPW, CH), jnp.int32),
          pltpu.VMEM((CH, DP), jnp.float32),
          pltpu.VMEM((CH, DP), jnp.float32),
          pltpu.SemaphoreType.DMA,
          pltpu.SemaphoreType.DMA,
      ],
  )(table, dst3)


def _scatter_body(tp_hbm, src_hbm, zsum_hbm, psum_hbm,
                  slab_v, rows_a, rows_b, zb_v, ssum, sem_a, sem_b):
  c = lax.axis_index("c")
  s = lax.axis_index("s")
  wid = s * NC + c

  pltpu.sync_copy(zsum_hbm, zb_v)
  pltpu.sync_copy(zb_v, ssum.at[pl.ds(s * RPS, RPS)])
  pltpu.sync_copy(src_hbm.at[:, wid], slab_v)
  plsc.subcore_barrier()

  def issue(t, buf, sem):
    g = t * NW + wid

    @pl.when(g < NCHT)
    def _():
      pltpu.async_copy(tp_hbm.at[pl.ds(g * CH, CH), pl.ds(0, DP)], buf, sem)

  def drain(t, buf, sem):
    g = t * NW + wid

    @pl.when(g < NCHT)
    def _():
      pltpu.make_async_copy(
          tp_hbm.at[pl.ds(g * CH, CH), pl.ds(0, DP)], buf, sem).wait()
      pltpu.sync_copy(buf, ssum.at[slab_v.at[t]], add=True)

  def step(j, carry):
    issue(2 * j, rows_a, sem_a)
    issue(2 * j + 1, rows_b, sem_b)
    drain(2 * j, rows_a, sem_a)
    drain(2 * j + 1, rows_b, sem_b)
    return carry

  lax.fori_loop(0, TPW // 2, step, 0)
  plsc.subcore_barrier()

  pltpu.sync_copy(ssum.at[pl.ds(s * RPS, RPS)], zb_v)
  pltpu.sync_copy(zb_v, psum_hbm.at[c, pl.ds(s * RPS, RPS), pl.ds(0, DP)])


def _sc_scatter(tp, src3, zsum):
  mesh = plsc.VectorSubcoreMesh(core_axis_name="c", subcore_axis_name="s")
  return pl.kernel(
      _scatter_body,
      out_type=jax.ShapeDtypeStruct((NC, NPAD, LW), jnp.float32),
      mesh=mesh,
      compiler_params=pltpu.CompilerParams(use_tc_tiling_on_sc=False),
      scratch_types=[
          pltpu.VMEM((TPW, CH), jnp.int32),
          pltpu.VMEM((CH, DP), jnp.float32),
          pltpu.VMEM((CH, DP), jnp.float32),
          pltpu.VMEM((RPS, DP), jnp.float32),
          pltpu.VMEM_SHARED((NPAD, DP), jnp.float32),
          pltpu.SemaphoreType.DMA,
          pltpu.SemaphoreType.DMA,
      ],
  )(tp, src3, zsum)


def _dense_body(ea_ref, sh_ref, x_ref, w1t_ref, b1_ref, w2t_ref, b2_ref,
                eyep_ref, tp_ref):
  f32 = jnp.float32
  eyep = eyep_ref[...]
  h = jnp.maximum(
      lax.dot_general(w1t_ref[...], ea_ref[...], (((1,), (0,)), ((), ())),
                      preferred_element_type=f32) + b1_ref[...], 0.0)
  w2d = lax.dot_general(w2t_ref[...], h, (((1,), (0,)), ((), ())),
                        preferred_element_type=f32) + b2_ref[...]
  shT = sh_ref[...]
  lane = lax.broadcasted_iota(jnp.int32, (1, LW), 1)
  xm = jnp.where(lane < DP, x_ref[...], 0.0)
  xt = lax.dot_general(eyep, xm, (((1,), (1,)), ((), ())),
                       preferred_element_type=f32)

  X = xt.reshape(DP, BS, 128)
  S = shT.reshape(4, BS, 128)
  W = w2d.reshape(WNUM, BS, 128)
  SH0 = S[0]
  SH1 = [S[1], S[2], S[3]]

  def XP(j):
    return X[j]

  def WP(r):
    return W[r]

  f0e = [XP(i) * SH0 for i in range(16)]
  f0e += [(XP(16 + 3 * i) * SH1[0] + XP(17 + 3 * i) * SH1[1]
           + XP(18 + 3 * i) * SH1[2]) * _INV3 for i in range(4)]

  o1o = [[XP(i) * SH1[cc] for cc in range(3)] for i in range(16)]
  o1o += [[XP(16 + 3 * i + cc) * SH0 for cc in range(3)] for i in range(4)]
  for i in range(4):
    a = [XP(28 + 3 * i + cc) for cc in range(3)]
    o1o.append([(a[(cc + 1) % 3] * SH1[(cc + 2) % 3]
                 - a[(cc + 2) % 3] * SH1[(cc + 1) % 3]) * _INV2
                for cc in range(3)])

  o1e = []
  for i in range(4):
    a = [XP(16 + 3 * i + cc) for cc in range(3)]
    o1e.append([(a[(cc + 1) % 3] * SH1[(cc + 2) % 3]
                 - a[(cc + 2) % 3] * SH1[(cc + 1) % 3]) * _INV2
                for cc in range(3)])
  o1e += [[XP(28 + 3 * i + cc) * SH0 for cc in range(3)] for i in range(4)]
  o1e += [[XP(40 + i) * SH1[cc] for cc in range(3)] for i in range(2)]

  f0o = [(XP(28 + 3 * i) * SH1[0] + XP(29 + 3 * i) * SH1[1]
          + XP(30 + 3 * i) * SH1[2]) * _INV3 for i in range(4)]
  f0o += [XP(40 + i) * SH0 for i in range(2)]

  planes = []
  for o in range(16):
    acc = f0e[0] * WP(o)
    for i in range(1, 20):
      acc += f0e[i] * WP(i * 16 + o)
    planes.append(acc)
  for o in range(4):
    for cc in range(3):
      acc = o1o[0][cc] * WP(320 + o)
      for i in range(1, 24):
        acc += o1o[i][cc] * WP(320 + i * 4 + o)
      planes.append(acc)
  for o in range(4):
    for cc in range(3):
      acc = o1e[0][cc] * WP(416 + o)
      for i in range(1, 10):
        acc += o1e[i][cc] * WP(416 + i * 4 + o)
      planes.append(acc)
  for o in range(2):
    acc = f0o[0] * WP(456 + o)
    for i in range(1, 6):
      acc += f0o[i] * WP(456 + i * 2 + o)
    planes.append(acc)

  zero = jnp.zeros_like(planes[0])
  planes += [jnp.ones_like(zero)] + [zero] * (DP - DIN - 1)
  tpt = jnp.stack(planes, axis=0).reshape(DP, BE)
  tp_ref[...] = lax.dot_general(tpt, eyep, (((0,), (0,)), ((), ())),
                                preferred_element_type=f32)


def _tc_dense(ea, sh, x, w1t, b1c, w2t, b2c, eyep):
  return pl.pallas_call(
      _dense_body,
      grid=(GRID,),
      in_specs=[
          pl.BlockSpec((DE, BE), lambda i: (0, i)),
          pl.BlockSpec((4, BE), lambda i: (0, i)),
          pl.BlockSpec((BE, LW), lambda i: (i, 0)),
          pl.BlockSpec((DE, DE), lambda i: (0, 0)),
          pl.BlockSpec((DE, 1), lambda i: (0, 0)),
          pl.BlockSpec((WNUM, DE), lambda i: (0, 0)),
          pl.BlockSpec((WNUM, 1), lambda i: (0, 0)),
          pl.BlockSpec((DP, LW), lambda i: (0, 0)),
      ],
      out_specs=pl.BlockSpec((BE, LW), lambda i: (i, 0)),
      out_shape=jax.ShapeDtypeStruct((E, LW), jnp.float32),
  )(ea, sh, x, w1t, b1c, w2t, b2c, eyep)


def _combine_body(ps_ref, na_ref, out_ref):
  sums = ps_ref[0] + ps_ref[1]
  cnt = sums[:N, DIN:DIN + 1]
  out_ref[...] = sums[:N, :DIN] / jnp.maximum(cnt, 1.0) + na_ref[...]


def _tc_combine(psum, node_attr):
  return pl.pallas_call(
      _combine_body,
      out_shape=jax.ShapeDtypeStruct((N, DIN), jnp.float32),
  )(psum, node_attr)


@jax.jit
def kernel(node_attr, edge_index, edge_attr, edge_sh,
           fc_w1, fc_b1, fc_w2, fc_b2):
  f32 = jnp.float32
  node_attr = node_attr.astype(f32)
  edge_src = edge_index[0].astype(jnp.int32)
  edge_dst = edge_index[1].astype(jnp.int32)

  table = jnp.pad(node_attr, ((0, 0), (0, DP - DIN)))
  dst3 = jnp.pad(edge_dst, (0, EPAD - E)).reshape(TPW, NW, CH)
  src3 = jnp.pad(edge_src, (0, EPAD - E)).reshape(TPW, NW, CH)

  scale = np.concatenate([
      np.full(320, 1.0 / np.sqrt(20.0)),
      np.full(96, 1.0 / np.sqrt(24.0)),
      np.full(40, 1.0 / np.sqrt(10.0)),
      np.full(12, 1.0 / np.sqrt(6.0)),
  ]).astype(np.float32)
  w1t = fc_w1.astype(f32).T
  b1c = fc_b1.astype(f32)[:, None]
  w2t = (fc_w2.astype(f32) * scale[None, :]).T
  b2c = (fc_b2.astype(f32) * scale)[:, None]
  eyep = jnp.eye(DP, LW, dtype=f32)

  x = _sc_gather(table, dst3)
  tp = _tc_dense(edge_attr.astype(f32).T, edge_sh.astype(f32).T, x,
                 w1t, b1c, w2t, b2c, eyep)

  zsum = jnp.zeros((RPS, DP), f32)
  psum = _sc_scatter(tp, src3, zsum)

  return _tc_combine(psum, node_attr)


# speedup vs baseline: 1.9542x; 1.0837x over previous
import functools

import jax
import jax.numpy as jnp
import numpy as np
from jax import lax
from jax.experimental import pallas as pl
from jax.experimental.pallas import tpu as pltpu
from jax.experimental.pallas import tpu_sc as plsc

N = 10000
E = 160000
DIN = 42
DP = 48
LW = 128
DE = 16
HID = 16
WNUM = 468

NC, NS = 2, 16
NW = NC * NS
CH = 128
NPAD = 10240
RPS = NPAD // NS

BE = 6400
BS = BE // 128

E1 = 76800
E2 = E - E1

_INV3 = float(1.0 / np.sqrt(3.0))
_INV2 = float(1.0 / np.sqrt(2.0))


def _tpw(ncht):
  t = -(-ncht // NW)
  return t + (t % 2)


def _gather_body(ncht, tpw, table_hbm, idx_hbm, out_hbm, slab_v,
                 rows_a, rows_b, sem_a, sem_b):
  c = lax.axis_index("c")
  s = lax.axis_index("s")
  wid = s * NC + c
  pltpu.sync_copy(idx_hbm.at[:, wid], slab_v)

  def issue(t, buf, sem):
    @pl.when(t * NW + wid < ncht)
    def _():
      pltpu.async_copy(table_hbm.at[slab_v.at[t]], buf, sem)

  def drain(t, buf, sem):
    g = t * NW + wid

    @pl.when(g < ncht)
    def _():
      pltpu.make_async_copy(table_hbm.at[slab_v.at[t]], buf, sem).wait()
      pltpu.sync_copy(buf, out_hbm.at[pl.ds(g * CH, CH), pl.ds(0, DP)])

  def step(j, carry):
    issue(2 * j, rows_a, sem_a)
    issue(2 * j + 1, rows_b, sem_b)
    drain(2 * j, rows_a, sem_a)
    drain(2 * j + 1, rows_b, sem_b)
    return carry

  lax.fori_loop(0, tpw // 2, step, 0)


def _sc_gather(table, dst3, e_cnt):
  ncht = e_cnt // CH
  tpw = dst3.shape[0]
  mesh = plsc.VectorSubcoreMesh(core_axis_name="c", subcore_axis_name="s")
  return pl.kernel(
      functools.partial(_gather_body, ncht, tpw),
      out_type=jax.ShapeDtypeStruct((e_cnt, LW), jnp.float32),
      mesh=mesh,
      compiler_params=pltpu.CompilerParams(use_tc_tiling_on_sc=False),
      scratch_types=[
          pltpu.VMEM((tpw, CH), jnp.int32),
          pltpu.VMEM((CH, DP), jnp.float32),
          pltpu.VMEM((CH, DP), jnp.float32),
          pltpu.SemaphoreType.DMA,
          pltpu.SemaphoreType.DMA,
      ],
  )(table, dst3)


def _scatter_body(ncht, tpw, tp_hbm, src_hbm, zsum_hbm, psum_hbm,
                  slab_v, rows_a, rows_b, zb_v, ssum, sem_a, sem_b):
  c = lax.axis_index("c")
  s = lax.axis_index("s")
  wid = s * NC + c

  pltpu.sync_copy(zsum_hbm, zb_v)
  pltpu.sync_copy(zb_v, ssum.at[pl.ds(s * RPS, RPS)])
  pltpu.sync_copy(src_hbm.at[:, wid], slab_v)
  plsc.subcore_barrier()

  def issue(t, buf, sem):
    g = t * NW + wid

    @pl.when(g < ncht)
    def _():
      pltpu.async_copy(tp_hbm.at[pl.ds(g * CH, CH), pl.ds(0, DP)], buf, sem)

  def drain(t, buf, sem):
    g = t * NW + wid

    @pl.when(g < ncht)
    def _():
      pltpu.make_async_copy(
          tp_hbm.at[pl.ds(g * CH, CH), pl.ds(0, DP)], buf, sem).wait()
      pltpu.sync_copy(buf, ssum.at[slab_v.at[t]], add=True)

  def step(j, carry):
    issue(2 * j, rows_a, sem_a)
    issue(2 * j + 1, rows_b, sem_b)
    drain(2 * j, rows_a, sem_a)
    drain(2 * j + 1, rows_b, sem_b)
    return carry

  lax.fori_loop(0, tpw // 2, step, 0)
  plsc.subcore_barrier()

  pltpu.sync_copy(ssum.at[pl.ds(s * RPS, RPS)], zb_v)
  pltpu.sync_copy(zb_v, psum_hbm.at[c, pl.ds(s * RPS, RPS), pl.ds(0, DP)])


def _sc_scatter(tp, src3, zsum):
  ncht = tp.shape[0] // CH
  tpw = src3.shape[0]
  mesh = plsc.VectorSubcoreMesh(core_axis_name="c", subcore_axis_name="s")
  return pl.kernel(
      functools.partial(_scatter_body, ncht, tpw),
      out_type=jax.ShapeDtypeStruct((NC, NPAD, LW), jnp.float32),
      mesh=mesh,
      compiler_params=pltpu.CompilerParams(use_tc_tiling_on_sc=False),
      scratch_types=[
          pltpu.VMEM((tpw, CH), jnp.int32),
          pltpu.VMEM((CH, DP), jnp.float32),
          pltpu.VMEM((CH, DP), jnp.float32),
          pltpu.VMEM((RPS, DP), jnp.float32),
          pltpu.VMEM_SHARED((NPAD, DP), jnp.float32),
          pltpu.SemaphoreType.DMA,
          pltpu.SemaphoreType.DMA,
      ],
  )(tp, src3, zsum)


def _dense_body(ea_ref, sh_ref, x_ref, w1t_ref, b1_ref, w2t_ref, b2_ref,
                eyep_ref, tp_ref):
  f32 = jnp.float32
  eyep = eyep_ref[...]
  h = jnp.maximum(
      lax.dot_general(w1t_ref[...], ea_ref[...], (((1,), (0,)), ((), ())),
                      preferred_element_type=f32) + b1_ref[...], 0.0)
  w2d = lax.dot_general(w2t_ref[...], h, (((1,), (0,)), ((), ())),
                        preferred_element_type=f32) + b2_ref[...]
  shT = sh_ref[...]
  lane = lax.broadcasted_iota(jnp.int32, (1, LW), 1)
  xm = jnp.where(lane < DP, x_ref[...], 0.0)
  xt = lax.dot_general(eyep, xm, (((1,), (1,)), ((), ())),
                       preferred_element_type=f32)

  X = xt.reshape(DP, BS, 128)
  S = shT.reshape(4, BS, 128)
  W = w2d.reshape(WNUM, BS, 128)
  SH0 = S[0]
  SH1 = [S[1], S[2], S[3]]

  def XP(j):
    return X[j]

  def WP(r):
    return W[r]

  f0e = [XP(i) * SH0 for i in range(16)]
  f0e += [(XP(16 + 3 * i) * SH1[0] + XP(17 + 3 * i) * SH1[1]
           + XP(18 + 3 * i) * SH1[2]) * _INV3 for i in range(4)]

  o1o = [[XP(i) * SH1[cc] for cc in range(3)] for i in range(16)]
  o1o += [[XP(16 + 3 * i + cc) * SH0 for cc in range(3)] for i in range(4)]
  for i in range(4):
    a = [XP(28 + 3 * i + cc) for cc in range(3)]
    o1o.append([(a[(cc + 1) % 3] * SH1[(cc + 2) % 3]
                 - a[(cc + 2) % 3] * SH1[(cc + 1) % 3]) * _INV2
                for cc in range(3)])

  o1e = []
  for i in range(4):
    a = [XP(16 + 3 * i + cc) for cc in range(3)]
    o1e.append([(a[(cc + 1) % 3] * SH1[(cc + 2) % 3]
                 - a[(cc + 2) % 3] * SH1[(cc + 1) % 3]) * _INV2
                for cc in range(3)])
  o1e += [[XP(28 + 3 * i + cc) * SH0 for cc in range(3)] for i in range(4)]
  o1e += [[XP(40 + i) * SH1[cc] for cc in range(3)] for i in range(2)]

  f0o = [(XP(28 + 3 * i) * SH1[0] + XP(29 + 3 * i) * SH1[1]
          + XP(30 + 3 * i) * SH1[2]) * _INV3 for i in range(4)]
  f0o += [XP(40 + i) * SH0 for i in range(2)]

  planes = []
  for o in range(16):
    acc = f0e[0] * WP(o)
    for i in range(1, 20):
      acc += f0e[i] * WP(i * 16 + o)
    planes.append(acc)
  for o in range(4):
    for cc in range(3):
      acc = o1o[0][cc] * WP(320 + o)
      for i in range(1, 24):
        acc += o1o[i][cc] * WP(320 + i * 4 + o)
      planes.append(acc)
  for o in range(4):
    for cc in range(3):
      acc = o1e[0][cc] * WP(416 + o)
      for i in range(1, 10):
        acc += o1e[i][cc] * WP(416 + i * 4 + o)
      planes.append(acc)
  for o in range(2):
    acc = f0o[0] * WP(456 + o)
    for i in range(1, 6):
      acc += f0o[i] * WP(456 + i * 2 + o)
    planes.append(acc)

  zero = jnp.zeros_like(planes[0])
  planes += [jnp.ones_like(zero)] + [zero] * (DP - DIN - 1)
  tpt = jnp.stack(planes, axis=0).reshape(DP, BE)
  tp_ref[...] = lax.dot_general(tpt, eyep, (((0,), (0,)), ((), ())),
                                preferred_element_type=f32)


def _tc_dense(eaT, shT, x, w1t, b1c, w2t, b2c, eyep, e_off, e_cnt):
  grid = e_cnt // BE
  boff = e_off // BE
  return pl.pallas_call(
      _dense_body,
      grid=(grid,),
      in_specs=[
          pl.BlockSpec((DE, BE), lambda i: (0, i + boff)),
          pl.BlockSpec((4, BE), lambda i: (0, i + boff)),
          pl.BlockSpec((BE, LW), lambda i: (i, 0)),
          pl.BlockSpec((DE, DE), lambda i: (0, 0)),
          pl.BlockSpec((DE, 1), lambda i: (0, 0)),
          pl.BlockSpec((WNUM, DE), lambda i: (0, 0)),
          pl.BlockSpec((WNUM, 1), lambda i: (0, 0)),
          pl.BlockSpec((DP, LW), lambda i: (0, 0)),
      ],
      out_specs=pl.BlockSpec((BE, LW), lambda i: (i, 0)),
      out_shape=jax.ShapeDtypeStruct((e_cnt, LW), jnp.float32),
  )(eaT, shT, x, w1t, b1c, w2t, b2c, eyep)


def _combine_body(p1_ref, p2_ref, na_ref, out_ref):
  sums = (p1_ref[0] + p1_ref[1]) + (p2_ref[0] + p2_ref[1])
  cnt = sums[:N, DIN:DIN + 1]
  out_ref[...] = sums[:N, :DIN] / jnp.maximum(cnt, 1.0) + na_ref[...]


def _tc_combine(psum1, psum2, node_attr):
  return pl.pallas_call(
      _combine_body,
      out_shape=jax.ShapeDtypeStruct((N, DIN), jnp.float32),
  )(psum1, psum2, node_attr)


def _slab(idx, e_cnt):
  ncht = e_cnt // CH
  tpw = _tpw(ncht)
  return jnp.pad(idx, (0, tpw * NW * CH - e_cnt)).reshape(tpw, NW, CH)


@jax.jit
def kernel(node_attr, edge_index, edge_attr, edge_sh,
           fc_w1, fc_b1, fc_w2, fc_b2):
  f32 = jnp.float32
  node_attr = node_attr.astype(f32)
  edge_src = edge_index[0].astype(jnp.int32)
  edge_dst = edge_index[1].astype(jnp.int32)

  table = jnp.pad(node_attr, ((0, 0), (0, DP - DIN)))

  scale = np.concatenate([
      np.full(320, 1.0 / np.sqrt(20.0)),
      np.full(96, 1.0 / np.sqrt(24.0)),
      np.full(40, 1.0 / np.sqrt(10.0)),
      np.full(12, 1.0 / np.sqrt(6.0)),
  ]).astype(np.float32)
  w1t = fc_w1.astype(f32).T
  b1c = fc_b1.astype(f32)[:, None]
  w2t = (fc_w2.astype(f32) * scale[None, :]).T
  b2c = (fc_b2.astype(f32) * scale)[:, None]
  eyep = jnp.eye(DP, LW, dtype=f32)
  eaT = edge_attr.astype(f32).T
  shT = edge_sh.astype(f32).T
  zsum = jnp.zeros((RPS, DP), f32)

  halves = []
  for e_off, e_cnt in ((0, E1), (E1, E2)):
    x = _sc_gather(table, _slab(edge_dst[e_off:e_off + e_cnt], e_cnt), e_cnt)
    tp = _tc_dense(eaT, shT, x, w1t, b1c, w2t, b2c, eyep, e_off, e_cnt)
    psum = _sc_scatter(tp, _slab(edge_src[e_off:e_off + e_cnt], e_cnt), zsum)
    halves.append(psum)

  return _tc_combine(halves[0], halves[1], node_attr)
